# baseline (speedup 1.0000x reference)
import jax
import jax.numpy as jnp
from jax.experimental import pallas as pl
from jax.experimental.pallas import tpu as pltpu

NUM_BLOCKS = 16
BLOCK_SIZE = 256
BLOCK_RANK = 256
TOP_K = 8

ROW_TILE = 2048


def _adapter_kernel(idx_ref, score_ref, x_ref, dw_ref, db_ref, uw_ref, ub_ref,
                    out_ref, w_scratch):
    e = pl.program_id(0)
    t = pl.program_id(1)
    rows = pl.ds(t * ROW_TILE, ROW_TILE)

    @pl.when(e == 0)
    def _compute_routing():
        idx = idx_ref[rows, :]
        score = score_ref[rows, :]
        m = jnp.max(score, axis=1, keepdims=True)
        ex = jnp.exp(score - m)
        sm = ex / jnp.sum(ex, axis=1, keepdims=True)
        cols = [
            jnp.sum(jnp.where(idx == b, sm, 0.0), axis=1, keepdims=True)
            for b in range(NUM_BLOCKS)
        ]
        w_scratch[rows, :] = jnp.concatenate(cols, axis=1)

    sub = jax.lax.broadcasted_iota(jnp.int32, (NUM_BLOCKS, BLOCK_SIZE), 0)
    onehot = (sub == e).astype(jnp.float32)
    w = jnp.dot(w_scratch[rows, :], onehot,
                preferred_element_type=jnp.float32)

    x = x_ref[...].astype(jnp.bfloat16)
    dw = dw_ref[0].astype(jnp.bfloat16)
    uw = uw_ref[0].astype(jnp.bfloat16)
    h = jnp.dot(x, dw,
                preferred_element_type=jnp.float32).astype(jnp.bfloat16)
    h = h + db_ref[0]
    hh = h * jnp.bfloat16(0.5)
    act = hh + hh * jnp.tanh(hh)
    out = jnp.dot(act, uw, preferred_element_type=jnp.float32) + ub_ref[0]
    out_ref[...] = out * w


@jax.jit
def kernel(hidden_states, active_idx, active_score, down_w, down_b, up_w, up_b):
    batch, seq_len, hidden = hidden_states.shape
    n_rows = batch * seq_len
    x2d = hidden_states.reshape(n_rows, hidden)
    n_tiles = n_rows // ROW_TILE

    grid = (NUM_BLOCKS, n_tiles)
    out = pl.pallas_call(
        _adapter_kernel,
        grid=grid,
        in_specs=[
            pl.BlockSpec((n_rows, TOP_K), lambda e, t: (0, 0)),
            pl.BlockSpec((n_rows, TOP_K), lambda e, t: (0, 0)),
            pl.BlockSpec((ROW_TILE, BLOCK_SIZE), lambda e, t: (t, e)),
            pl.BlockSpec((1, BLOCK_SIZE, BLOCK_RANK), lambda e, t: (e, 0, 0)),
            pl.BlockSpec((1, 1, BLOCK_RANK), lambda e, t: (e, 0, 0)),
            pl.BlockSpec((1, BLOCK_RANK, BLOCK_SIZE), lambda e, t: (e, 0, 0)),
            pl.BlockSpec((1, 1, BLOCK_SIZE), lambda e, t: (e, 0, 0)),
        ],
        out_specs=pl.BlockSpec((ROW_TILE, BLOCK_SIZE), lambda e, t: (t, e)),
        out_shape=jax.ShapeDtypeStruct((n_rows, hidden), jnp.float32),
        scratch_shapes=[pltpu.VMEM((n_rows, NUM_BLOCKS), jnp.float32)],
    )(active_idx, active_score, x2d, down_w,
      down_b.reshape(NUM_BLOCKS, 1, BLOCK_RANK).astype(jnp.bfloat16), up_w,
      up_b.reshape(NUM_BLOCKS, 1, BLOCK_SIZE))
    return out.reshape(batch, seq_len, hidden)
